# Initial kernel scaffold; baseline (speedup 1.0000x reference)
import functools

import jax
import jax.numpy as jnp
from jax import lax
from jax.experimental import pallas as pl
from jax.experimental.pallas import tpu as pltpu
from jax.experimental.pallas import tpu_sc as plsc

NUM_USER = 25000
NUM_ITEM = 25000
N_NODES = NUM_USER + NUM_ITEM
N_EDGES = 800000
DIM_FEAT = 128
DIM = 64
HALF = 32

BR = 1000
NB = NUM_USER // BR


def _dense_body(pref_ref, feat_ref, w_ref, b_ref, cw_ref, out_ref):
    g = pl.program_id(0)
    t = lax.dot_general(feat_ref[...], w_ref[...],
                        (((1,), (1,)), ((), ())),
                        preferred_element_type=jnp.float32) + b_ref[...]
    x = jnp.where(g == 0, pref_ref[...], t)
    norm = jnp.sqrt(jnp.sum(x * x, axis=1, keepdims=True))
    x = x / jnp.maximum(norm, 1e-12)
    y = lax.dot_general(x, cw_ref[...], (((1,), (0,)), ((), ())),
                        preferred_element_type=jnp.float32)
    out_ref[0, :, :] = y[:, :HALF]
    out_ref[1, :, :] = y[:, HALF:]


_dense = pl.pallas_call(
    _dense_body,
    grid=(2, NB),
    in_specs=[
        pl.BlockSpec((BR, DIM), lambda g, j: (j, 0)),
        pl.BlockSpec((BR, DIM_FEAT), lambda g, j: (j, 0)),
        pl.BlockSpec((DIM, DIM_FEAT), lambda g, j: (0, 0)),
        pl.BlockSpec((1, DIM), lambda g, j: (0, 0)),
        pl.BlockSpec((DIM, DIM), lambda g, j: (0, 0)),
    ],
    out_specs=pl.BlockSpec((2, BR, HALF), lambda g, j: (0, g * NB + j, 0)),
    out_shape=jax.ShapeDtypeStruct((2, N_NODES, HALF), jnp.float32),
)

NC = 2
NS = 16
SUB = 128
SUBS = 16
OUTER = 25
EROWS_PER_TILE = OUTER * SUBS
NE_PAD = NS * EROWS_PER_TILE * SUB
EROWS = NE_PAD // SUB
CH = SUBS * SUB
ACC_ROWS = N_NODES + 8
RPT = N_NODES // NS
DRAIN = 625
NDRAIN = RPT // DRAIN

_mesh = plsc.VectorSubcoreMesh(core_axis_name="c", subcore_axis_name="s",
                               num_cores=NC, num_subcores=NS)


@functools.partial(
    pl.kernel,
    out_type=jax.ShapeDtypeStruct((N_NODES, DIM), jnp.float32),
    mesh=_mesh,
    scratch_types=[
        pltpu.VMEM((SUBS, SUB), jnp.int32),
        pltpu.VMEM((SUBS, SUB), jnp.int32),
        pltpu.VMEM((CH, HALF), jnp.float32),
        pltpu.VMEM_SHARED((ACC_ROWS, HALF), jnp.float32),
        pltpu.SemaphoreType.DMA,
    ],
)
def _sc_agg(xw_hbm, src_hbm, dst_hbm, out_hbm, src_v, dst_v, rows_v, acc_sh, sem):
    cid = lax.axis_index("c")
    sid = lax.axis_index("s")
    z16 = jnp.zeros((16,), jnp.float32)

    def _zrow(i, _):
        rows_v[i, 0:16] = z16
        rows_v[i, 16:32] = z16
        return 0

    lax.fori_loop(0, CH, _zrow, 0)
    abase = sid * RPT
    pltpu.sync_copy(rows_v, acc_sh.at[pl.ds(abase, CH)])
    pltpu.sync_copy(rows_v.at[pl.ds(0, RPT - CH)],
                    acc_sh.at[pl.ds(abase + CH, RPT - CH)])
    plsc.subcore_barrier()

    ebase = sid * EROWS_PER_TILE

    def _edge_iter(o, _):
        r0 = ebase + o * SUBS
        pltpu.sync_copy(src_hbm.at[pl.ds(r0, SUBS)], src_v)
        pltpu.sync_copy(dst_hbm.at[pl.ds(r0, SUBS)], dst_v)
        descs = []
        for j in range(SUBS):
            descs.append(pltpu.async_copy(
                xw_hbm.at[cid].at[src_v.at[j]],
                rows_v.at[pl.ds(j * SUB, SUB)], sem))
        for d in descs:
            d.wait()
        for j in range(SUBS):
            pltpu.sync_copy(rows_v.at[pl.ds(j * SUB, SUB)],
                            acc_sh.at[dst_v.at[j]], add=True)
        return 0

    lax.fori_loop(0, OUTER, _edge_iter, 0)
    plsc.subcore_barrier()

    def _drain(k, _):
        r0 = abase + k * DRAIN
        pltpu.sync_copy(acc_sh.at[pl.ds(r0, DRAIN)], rows_v.at[pl.ds(0, DRAIN)])

        def _lr(i, _):
            v0 = rows_v[i, 0:16]
            rows_v[i, 0:16] = jnp.maximum(v0, v0 * 0.01)
            v1 = rows_v[i, 16:32]
            rows_v[i, 16:32] = jnp.maximum(v1, v1 * 0.01)
            return 0

        lax.fori_loop(0, DRAIN, _lr, 0)
        pltpu.sync_copy(rows_v.at[pl.ds(0, DRAIN)],
                        out_hbm.at[pl.ds(r0, DRAIN), pl.ds(cid * HALF, HALF)])
        return 0

    lax.fori_loop(0, NDRAIN, _drain, 0)


def kernel(features, edge_index, id_embedding, preference, MLP_W, MLP_b, conv_W):
    del id_embedding
    xw = _dense(preference, features, MLP_W, MLP_b.reshape(1, DIM), conv_W)
    src = edge_index[0].astype(jnp.int32)
    dst = edge_index[1].astype(jnp.int32)
    pad = NE_PAD - N_EDGES
    src = jnp.concatenate([src, jnp.zeros((pad,), jnp.int32)]).reshape(EROWS, SUB)
    dst = jnp.concatenate([dst, jnp.full((pad,), N_NODES, jnp.int32)]).reshape(EROWS, SUB)
    return _sc_agg(xw, src, dst)


# speedup vs baseline: 5.1100x; 5.1100x over previous
import functools

import jax
import jax.numpy as jnp
from jax import lax
from jax.experimental import pallas as pl
from jax.experimental.pallas import tpu as pltpu
from jax.experimental.pallas import tpu_sc as plsc

NUM_USER = 25000
NUM_ITEM = 25000
N_NODES = NUM_USER + NUM_ITEM
N_EDGES = 800000
DIM_FEAT = 128
DIM = 64
HALF = 32

BR = 1000
NB = NUM_USER // BR


def _dense_body(pref_ref, feat_ref, w_ref, b_ref, cw_ref, out_ref):
    g = pl.program_id(0)
    t = lax.dot_general(feat_ref[...], w_ref[...],
                        (((1,), (1,)), ((), ())),
                        preferred_element_type=jnp.float32) + b_ref[...]
    x = jnp.where(g == 0, pref_ref[...], t)
    norm = jnp.sqrt(jnp.sum(x * x, axis=1, keepdims=True))
    x = x / jnp.maximum(norm, 1e-12)
    y = lax.dot_general(x, cw_ref[...], (((1,), (0,)), ((), ())),
                        preferred_element_type=jnp.float32)
    out_ref[0, :, :] = y[:, :HALF]
    out_ref[1, :, :] = y[:, HALF:]


_dense = pl.pallas_call(
    _dense_body,
    grid=(2, NB),
    in_specs=[
        pl.BlockSpec((BR, DIM), lambda g, j: (j, 0)),
        pl.BlockSpec((BR, DIM_FEAT), lambda g, j: (j, 0)),
        pl.BlockSpec((DIM, DIM_FEAT), lambda g, j: (0, 0)),
        pl.BlockSpec((1, DIM), lambda g, j: (0, 0)),
        pl.BlockSpec((DIM, DIM), lambda g, j: (0, 0)),
    ],
    out_specs=pl.BlockSpec((2, BR, HALF), lambda g, j: (0, g * NB + j, 0)),
    out_shape=jax.ShapeDtypeStruct((2, N_NODES, HALF), jnp.float32),
)

NC = 2
NS = 16
SUB = 128
SUBS = 4
OUTER = 100
EROWS_PER_TILE = OUTER * SUBS
NE_PAD = NS * EROWS_PER_TILE * SUB
EROWS = NE_PAD // SUB
CH = SUBS * SUB
RPT = 3200
ACC_ROWS = NS * RPT
DRAIN = 320
NDRAIN = RPT // DRAIN

def _sc_body(xw_hbm, src_hbm, dst_hbm, out_hbm, src_v, dst_v, rows_v, acc_sh, sem):
    cid = lax.axis_index("c")
    sid = lax.axis_index("s")
    z16 = jnp.zeros((16,), jnp.float32)

    def _zrow(i, _):
        rows_v[i, 0:16] = z16
        rows_v[i, 16:32] = z16
        return 0

    lax.fori_loop(0, CH, _zrow, 0)
    abase = sid * RPT

    def _zchunk(k, _):
        pltpu.sync_copy(rows_v.at[pl.ds(0, DRAIN)],
                        acc_sh.at[pl.ds(abase + k * DRAIN, DRAIN)])
        return 0

    lax.fori_loop(0, NDRAIN, _zchunk, 0)
    plsc.subcore_barrier()

    ebase = sid * EROWS_PER_TILE

    def _edge_iter(o, _):
        r0 = ebase + o * SUBS
        pltpu.sync_copy(src_hbm.at[pl.ds(r0, SUBS)], src_v)
        pltpu.sync_copy(dst_hbm.at[pl.ds(r0, SUBS)], dst_v)
        descs = []
        for j in range(SUBS):
            descs.append(pltpu.async_copy(
                xw_hbm.at[cid].at[src_v.at[j]],
                rows_v.at[pl.ds(j * SUB, SUB)], sem))
        for d in descs:
            d.wait()
        for j in range(SUBS):
            pltpu.sync_copy(rows_v.at[pl.ds(j * SUB, SUB)],
                            acc_sh.at[dst_v.at[j]], add=True)
        return 0

    lax.fori_loop(0, OUTER, _edge_iter, 0)
    plsc.subcore_barrier()

    def _drain(k, _):
        r0 = abase + k * DRAIN
        pltpu.sync_copy(acc_sh.at[pl.ds(r0, DRAIN)], rows_v.at[pl.ds(0, DRAIN)])

        def _lr(i, _):
            v0 = rows_v[i, 0:16]
            rows_v[i, 0:16] = jnp.maximum(v0, v0 * 0.01)
            v1 = rows_v[i, 16:32]
            rows_v[i, 16:32] = jnp.maximum(v1, v1 * 0.01)
            return 0

        lax.fori_loop(0, DRAIN, _lr, 0)
        pltpu.sync_copy(rows_v.at[pl.ds(0, DRAIN)],
                        out_hbm.at[cid].at[pl.ds(r0, DRAIN)])
        return 0

    lax.fori_loop(0, NDRAIN, _drain, 0)


@functools.cache
def _make_sc_agg():
    mesh = plsc.VectorSubcoreMesh(core_axis_name="c", subcore_axis_name="s",
                                  num_cores=NC, num_subcores=NS)
    return pl.kernel(
        _sc_body,
        out_type=jax.ShapeDtypeStruct((NC, ACC_ROWS, HALF), jnp.float32),
        mesh=mesh,
        scratch_types=[
            pltpu.VMEM((SUBS, SUB), jnp.int32),
            pltpu.VMEM((SUBS, SUB), jnp.int32),
            pltpu.VMEM((CH, HALF), jnp.float32),
            pltpu.VMEM_SHARED((ACC_ROWS, HALF), jnp.float32),
            pltpu.SemaphoreType.DMA,
        ],
        compiler_params=pltpu.CompilerParams(use_tc_tiling_on_sc=False),
    )


def kernel(features, edge_index, id_embedding, preference, MLP_W, MLP_b, conv_W):
    del id_embedding
    xw = _dense(preference, features, MLP_W, MLP_b.reshape(1, DIM), conv_W)
    src = edge_index[0].astype(jnp.int32)
    dst = edge_index[1].astype(jnp.int32)
    pad = NE_PAD - N_EDGES
    src = jnp.concatenate([src, jnp.zeros((pad,), jnp.int32)]).reshape(EROWS, SUB)
    dst = jnp.concatenate([dst, jnp.full((pad,), N_NODES, jnp.int32)]).reshape(EROWS, SUB)
    out = _make_sc_agg()(xw, src, dst)
    return jnp.concatenate([out[0, :N_NODES], out[1, :N_NODES]], axis=1)


# speedup vs baseline: 7.4267x; 1.4534x over previous
import functools

import jax
import jax.numpy as jnp
from jax import lax
from jax.experimental import pallas as pl
from jax.experimental.pallas import tpu as pltpu
from jax.experimental.pallas import tpu_sc as plsc

NUM_USER = 25000
NUM_ITEM = 25000
N_NODES = NUM_USER + NUM_ITEM
N_EDGES = 800000
DIM_FEAT = 128
DIM = 64
HALF = 32

BR = 1000
NB = NUM_USER // BR


def _dense_body(pref_ref, feat_ref, w_ref, b_ref, cw_ref, out_ref):
    g = pl.program_id(0)
    t = lax.dot_general(feat_ref[...], w_ref[...],
                        (((1,), (1,)), ((), ())),
                        preferred_element_type=jnp.float32) + b_ref[...]
    x = jnp.where(g == 0, pref_ref[...], t)
    norm = jnp.sqrt(jnp.sum(x * x, axis=1, keepdims=True))
    x = x / jnp.maximum(norm, 1e-12)
    y = lax.dot_general(x, cw_ref[...], (((1,), (0,)), ((), ())),
                        preferred_element_type=jnp.float32)
    out_ref[0, :, :] = y[:, :HALF]
    out_ref[1, :, :] = y[:, HALF:]


_dense = pl.pallas_call(
    _dense_body,
    grid=(2, NB),
    in_specs=[
        pl.BlockSpec((BR, DIM), lambda g, j: (j, 0)),
        pl.BlockSpec((BR, DIM_FEAT), lambda g, j: (j, 0)),
        pl.BlockSpec((DIM, DIM_FEAT), lambda g, j: (0, 0)),
        pl.BlockSpec((1, DIM), lambda g, j: (0, 0)),
        pl.BlockSpec((DIM, DIM), lambda g, j: (0, 0)),
    ],
    out_specs=pl.BlockSpec((2, BR, HALF), lambda g, j: (0, g * NB + j, 0)),
    out_shape=jax.ShapeDtypeStruct((2, N_NODES, HALF), jnp.float32),
)

NC = 2
NS = 16
SUB = 128
SUBS = 3
CH = SUBS * SUB
OUTER = 132
EROWS_PER_TILE = OUTER * SUBS
NE_PAD = NS * EROWS_PER_TILE * SUB
EROWS = NE_PAD // SUB
RPT = 3136
ACC_ROWS = NS * RPT
DRAIN = 224
NDRAIN = RPT // DRAIN

def _sc_body(xw_hbm, ed_hbm, out_hbm, ed_a, ed_b, rows_a, rows_b, acc_sh,
             sem_ga, sem_gb, sem_sa, sem_sb):
    cid = lax.axis_index("c")
    sid = lax.axis_index("s")
    z16 = jnp.zeros((16,), jnp.float32)
    xw_c = xw_hbm.at[cid]

    def _gather(ed_v, rows_v, sem):
        for j in range(SUBS):
            pltpu.async_copy(xw_c.at[ed_v.at[j, 0]],
                             rows_v.at[pl.ds(j * SUB, SUB)], sem)

    def _gather_wait(ed_v, rows_v, sem):
        for j in range(SUBS):
            pltpu.make_async_copy(xw_c.at[ed_v.at[j, 0]],
                                  rows_v.at[pl.ds(j * SUB, SUB)], sem).wait()

    def _scatter(ed_v, rows_v, sem):
        for j in range(SUBS):
            pltpu.async_copy(rows_v.at[pl.ds(j * SUB, SUB)],
                             acc_sh.at[ed_v.at[j, 1]], sem, add=True)

    def _scatter_wait(ed_v, rows_v, sem):
        for j in range(SUBS):
            pltpu.make_async_copy(rows_v.at[pl.ds(j * SUB, SUB)],
                                  acc_sh.at[ed_v.at[j, 1]], sem).wait()

    def _zrow(i, _):
        rows_a[i, 0:16] = z16
        rows_a[i, 16:32] = z16
        return 0

    lax.fori_loop(0, DRAIN, _zrow, 0)
    abase = sid * RPT

    def _zchunk(k, _):
        pltpu.sync_copy(rows_a.at[pl.ds(0, DRAIN)],
                        acc_sh.at[pl.ds(abase + k * DRAIN, DRAIN)])
        return 0

    lax.fori_loop(0, NDRAIN, _zchunk, 0)
    plsc.subcore_barrier()

    ebase = sid * EROWS_PER_TILE

    def _edge_iter(o, _):
        ra = ebase + (2 * o) * SUBS
        rb = ra + SUBS

        @pl.when(o > 0)
        def _():
            _scatter_wait(ed_a, rows_a, sem_sa)

        pltpu.sync_copy(ed_hbm.at[pl.ds(ra, SUBS)], ed_a)
        _gather(ed_a, rows_a, sem_ga)

        @pl.when(o > 0)
        def _():
            _gather_wait(ed_b, rows_b, sem_gb)
            _scatter(ed_b, rows_b, sem_sb)

        @pl.when(o > 0)
        def _():
            _scatter_wait(ed_b, rows_b, sem_sb)

        pltpu.sync_copy(ed_hbm.at[pl.ds(rb, SUBS)], ed_b)
        _gather(ed_b, rows_b, sem_gb)
        _gather_wait(ed_a, rows_a, sem_ga)
        _scatter(ed_a, rows_a, sem_sa)
        return 0

    lax.fori_loop(0, OUTER // 2, _edge_iter, 0)
    _gather_wait(ed_b, rows_b, sem_gb)
    _scatter(ed_b, rows_b, sem_sb)
    _scatter_wait(ed_a, rows_a, sem_sa)
    _scatter_wait(ed_b, rows_b, sem_sb)
    plsc.subcore_barrier()

    def _drain(k, _):
        r0 = abase + k * DRAIN
        pltpu.sync_copy(acc_sh.at[pl.ds(r0, DRAIN)], rows_a.at[pl.ds(0, DRAIN)])

        def _lr(i, _):
            v0 = rows_a[i, 0:16]
            rows_a[i, 0:16] = jnp.maximum(v0, v0 * 0.01)
            v1 = rows_a[i, 16:32]
            rows_a[i, 16:32] = jnp.maximum(v1, v1 * 0.01)
            return 0

        lax.fori_loop(0, DRAIN, _lr, 0)
        pltpu.sync_copy(rows_a.at[pl.ds(0, DRAIN)],
                        out_hbm.at[cid].at[pl.ds(r0, DRAIN)])
        return 0

    lax.fori_loop(0, NDRAIN, _drain, 0)


@functools.cache
def _make_sc_agg():
    mesh = plsc.VectorSubcoreMesh(core_axis_name="c", subcore_axis_name="s",
                                  num_cores=NC, num_subcores=NS)
    return pl.kernel(
        _sc_body,
        out_type=jax.ShapeDtypeStruct((NC, ACC_ROWS, HALF), jnp.float32),
        mesh=mesh,
        scratch_types=[
            pltpu.VMEM((SUBS, 2, SUB), jnp.int32),
            pltpu.VMEM((SUBS, 2, SUB), jnp.int32),
            pltpu.VMEM((CH, HALF), jnp.float32),
            pltpu.VMEM((CH, HALF), jnp.float32),
            pltpu.VMEM_SHARED((ACC_ROWS, HALF), jnp.float32),
            pltpu.SemaphoreType.DMA,
            pltpu.SemaphoreType.DMA,
            pltpu.SemaphoreType.DMA,
            pltpu.SemaphoreType.DMA,
        ],
        compiler_params=pltpu.CompilerParams(use_tc_tiling_on_sc=False),
    )


def kernel(features, edge_index, id_embedding, preference, MLP_W, MLP_b, conv_W):
    del id_embedding
    xw = _dense(preference, features, MLP_W, MLP_b.reshape(1, DIM), conv_W)
    pad = NE_PAD - N_EDGES
    src = jnp.concatenate(
        [edge_index[0].astype(jnp.int32), jnp.zeros((pad,), jnp.int32)])
    dst = jnp.concatenate(
        [edge_index[1].astype(jnp.int32), jnp.full((pad,), N_NODES, jnp.int32)])
    ed = jnp.stack([src.reshape(EROWS, SUB), dst.reshape(EROWS, SUB)], axis=1)
    out = _make_sc_agg()(xw, ed)
    return jnp.concatenate([out[0, :N_NODES], out[1, :N_NODES]], axis=1)


# speedup vs baseline: 10.4740x; 1.4103x over previous
import functools

import jax
import jax.numpy as jnp
from jax import lax
from jax.experimental import pallas as pl
from jax.experimental.pallas import tpu as pltpu
from jax.experimental.pallas import tpu_sc as plsc

NUM_USER = 25000
NUM_ITEM = 25000
N_NODES = NUM_USER + NUM_ITEM
N_EDGES = 800000
DIM_FEAT = 128
DIM = 64
HALF = 32

BR = 1000
NB = NUM_USER // BR


def _dense_body(pref_ref, feat_ref, w_ref, b_ref, cw_ref, out_ref):
    g = pl.program_id(0)
    t = lax.dot_general(feat_ref[...], w_ref[...],
                        (((1,), (1,)), ((), ())),
                        preferred_element_type=jnp.float32) + b_ref[...]
    x = jnp.where(g == 0, pref_ref[...], t)
    norm = jnp.sqrt(jnp.sum(x * x, axis=1, keepdims=True))
    x = x / jnp.maximum(norm, 1e-12)
    y = lax.dot_general(x, cw_ref[...], (((1,), (0,)), ((), ())),
                        preferred_element_type=jnp.float32)
    out_ref[0, :, :] = y[:, :HALF]
    out_ref[1, :, :] = y[:, HALF:]


_dense = pl.pallas_call(
    _dense_body,
    grid=(2, NB),
    in_specs=[
        pl.BlockSpec((BR, DIM), lambda g, j: (jnp.where(g == 0, j, 0), 0)),
        pl.BlockSpec((BR, DIM_FEAT), lambda g, j: (jnp.where(g == 0, 0, j), 0)),
        pl.BlockSpec((DIM, DIM_FEAT), lambda g, j: (0, 0)),
        pl.BlockSpec((1, DIM), lambda g, j: (0, 0)),
        pl.BlockSpec((DIM, DIM), lambda g, j: (0, 0)),
    ],
    out_specs=pl.BlockSpec((2, BR, HALF), lambda g, j: (0, g * NB + j, 0)),
    out_shape=jax.ShapeDtypeStruct((2, N_NODES, HALF), jnp.float32),
)

NC = 2
NS = 16
SUB = 128
SUBS = 2
CH = SUBS * SUB
EROWS = N_EDGES // SUB
ERPT = 390
N_CH = ERPT // SUBS
NSLOT = N_CH + 3
EXTRA_BASE = NS * ERPT
RPT = 3136
ACC_ROWS = NS * RPT
DRAIN = 224
NDRAIN = RPT // DRAIN
LAST_R0 = 49952


def _sc_body(xw_hbm, src_hbm, dst_hbm, out_hbm,
             src_v, dst_v, rows_v, acc_sh, gsem, ssem, isem):
    cid = lax.axis_index("c")
    sid = lax.axis_index("s")
    z16 = jnp.zeros((16,), jnp.float32)
    xw_c = xw_hbm.at[cid]

    def _idx_load(c, r, wait):
        for ref, hbm in ((src_v[r], src_hbm), (dst_v[r], dst_hbm)):
            cp = pltpu.make_async_copy(
                hbm.at[pl.ds(sid * ERPT + c * SUBS, SUBS)], ref, isem[r])
            cp.wait() if wait else cp.start()

    def _gather(r, wait):
        for j in range(SUBS):
            cp = pltpu.make_async_copy(
                xw_c.at[src_v[r].at[j]],
                rows_v[r].at[pl.ds(j * SUB, SUB)], gsem[r])
            cp.wait() if wait else cp.start()

    def _scatter(r, wait):
        for j in range(SUBS):
            cp = pltpu.make_async_copy(
                rows_v[r].at[pl.ds(j * SUB, SUB)],
                acc_sh.at[dst_v[r].at[j]], ssem[r])
            cp.wait() if wait else cp.start(add=True)

    def _zrow(i, _):
        rows_v[0][i, 0:16] = z16
        rows_v[0][i, 16:32] = z16
        return 0

    lax.fori_loop(0, DRAIN, _zrow, 0)
    abase = sid * RPT

    def _zchunk(k, _):
        pltpu.sync_copy(rows_v[0].at[pl.ds(0, DRAIN)],
                        acc_sh.at[pl.ds(abase + k * DRAIN, DRAIN)])
        return 0

    lax.fori_loop(0, NDRAIN, _zchunk, 0)
    plsc.subcore_barrier()

    _idx_load(0, 0, wait=False)
    _idx_load(0, 0, wait=True)
    _gather(0, wait=False)

    def _edge_iter(o, _):
        for k in range(3):
            c = 3 * o + k

            @pl.when((c >= 2) & (c <= N_CH + 1))
            def _():
                _scatter((k + 1) % 3, wait=True)

            @pl.when(c + 1 < N_CH)
            def _():
                _idx_load(c + 1, (k + 1) % 3, wait=False)

            @pl.when(c < N_CH)
            def _():
                _gather(k, wait=True)
                _scatter(k, wait=False)

            @pl.when(c + 1 < N_CH)
            def _():
                _idx_load(c + 1, (k + 1) % 3, wait=True)
                _gather((k + 1) % 3, wait=False)
        return 0

    lax.fori_loop(0, NSLOT // 3, _edge_iter, 0)

    @pl.when(sid < EROWS - EXTRA_BASE)
    def _():
        for ref, hbm in ((src_v[0], src_hbm), (dst_v[0], dst_hbm)):
            pltpu.sync_copy(hbm.at[pl.ds(EXTRA_BASE + sid, 1)],
                            ref.at[pl.ds(0, 1)])
        pltpu.make_async_copy(xw_c.at[src_v[0].at[0]],
                              rows_v[0].at[pl.ds(0, SUB)], gsem[0]).start()
        pltpu.make_async_copy(xw_c.at[src_v[0].at[0]],
                              rows_v[0].at[pl.ds(0, SUB)], gsem[0]).wait()
        pltpu.make_async_copy(rows_v[0].at[pl.ds(0, SUB)],
                              acc_sh.at[dst_v[0].at[0]], ssem[0]).start(add=True)
        pltpu.make_async_copy(rows_v[0].at[pl.ds(0, SUB)],
                              acc_sh.at[dst_v[0].at[0]], ssem[0]).wait()

    plsc.subcore_barrier()

    def _drain(k, _):
        r0 = abase + k * DRAIN
        pltpu.sync_copy(acc_sh.at[pl.ds(r0, DRAIN)],
                        rows_v[0].at[pl.ds(0, DRAIN)])

        def _lr(i, _):
            v0 = rows_v[0][i, 0:16]
            rows_v[0][i, 0:16] = jnp.maximum(v0, v0 * 0.01)
            v1 = rows_v[0][i, 16:32]
            rows_v[0][i, 16:32] = jnp.maximum(v1, v1 * 0.01)
            return 0

        lax.fori_loop(0, DRAIN, _lr, 0)

        @pl.when(r0 <= N_NODES - DRAIN)
        def _():
            pltpu.sync_copy(
                rows_v[0].at[pl.ds(0, DRAIN)],
                out_hbm.at[pl.ds(r0, DRAIN), pl.ds(cid * HALF, HALF)])

        @pl.when(r0 == LAST_R0)
        def _():
            pltpu.sync_copy(
                rows_v[0].at[pl.ds(0, N_NODES - LAST_R0)],
                out_hbm.at[pl.ds(LAST_R0, N_NODES - LAST_R0),
                           pl.ds(cid * HALF, HALF)])
        return 0

    lax.fori_loop(0, NDRAIN, _drain, 0)


@functools.cache
def _make_sc_agg():
    mesh = plsc.VectorSubcoreMesh(core_axis_name="c", subcore_axis_name="s",
                                  num_cores=NC, num_subcores=NS)
    return pl.kernel(
        _sc_body,
        out_type=jax.ShapeDtypeStruct((N_NODES, DIM), jnp.float32),
        mesh=mesh,
        scratch_types=[
            [pltpu.VMEM((SUBS, SUB), jnp.int32) for _ in range(3)],
            [pltpu.VMEM((SUBS, SUB), jnp.int32) for _ in range(3)],
            [pltpu.VMEM((CH, HALF), jnp.float32) for _ in range(3)],
            pltpu.VMEM_SHARED((ACC_ROWS, HALF), jnp.float32),
            [pltpu.SemaphoreType.DMA for _ in range(3)],
            [pltpu.SemaphoreType.DMA for _ in range(3)],
            [pltpu.SemaphoreType.DMA for _ in range(3)],
        ],
        compiler_params=pltpu.CompilerParams(use_tc_tiling_on_sc=False),
    )


def kernel(features, edge_index, id_embedding, preference, MLP_W, MLP_b, conv_W):
    del id_embedding
    xw = _dense(preference, features, MLP_W, MLP_b.reshape(1, DIM), conv_W)
    src = edge_index[0].astype(jnp.int32).reshape(EROWS, SUB)
    dst = edge_index[1].astype(jnp.int32).reshape(EROWS, SUB)
    return _make_sc_agg()(xw, src, dst)
